# baseline (speedup 1.0000x reference)
import functools

import jax
import jax.numpy as jnp
from jax import lax
from jax.experimental import pallas as pl
from jax.experimental.pallas import tpu as pltpu
from jax.experimental.pallas import tpu_sc as plsc

N = 10000
E = 320000
D = 128

NC = 2
NS = 16
HALF = 5120
ACC_H = HALF + 8
RPT = HALF // NS
EPT = E // NS
STAGE = 400
NSTAGE = EPT // STAGE
SRING = 4
GPS = STAGE // 16
K = 80
RING = 2
CAP = 3840
DRAIN_AT = CAP - 3 * STAGE


def _linear_body(x_ref, wt_ref, b_ref, o_ref):
    o_ref[...] = (
        jnp.dot(x_ref[...], wt_ref[...], preferred_element_type=jnp.float32)
        + b_ref[...]
    )


_BM = 1000

_linear = pl.pallas_call(
    _linear_body,
    grid=(N // _BM,),
    in_specs=[
        pl.BlockSpec((_BM, D), lambda i: (i, 0)),
        pl.BlockSpec((D, D), lambda i: (0, 0)),
        pl.BlockSpec((1, D), lambda i: (0, 0)),
    ],
    out_specs=pl.BlockSpec((_BM, D), lambda i: (i, 0)),
    out_shape=jax.ShapeDtypeStruct((N, D), jnp.float32),
)


def _spmm_body(out_hbm, row_hbm, col_hbm, val_hbm, res_hbm,
               crow, ccol, cval,
               rrow0, rcol0, rval0, rrow1, rcol1, rval1,
               rrow2, rcol2, rval2, rrow3, rcol3, rval3,
               g0, g1, s0, s1, acc,
               semr0, semr1, semr2, semr3,
               semg0, semg1, sems0, sems1):
    cid = lax.axis_index("c")
    sid = lax.axis_index("s")
    lo = cid * HALF

    rstage = ((rrow0, rcol0, rval0), (rrow1, rcol1, rval1),
              (rrow2, rcol2, rval2), (rrow3, rcol3, rval3))
    semr = (semr0, semr1, semr2, semr3)
    gbuf = (g0, g1)
    sbuf = (s0, s1)
    semg = (semg0, semg1)
    sems = (sems0, sems1)

    zero16 = jnp.zeros((16,), jnp.float32)

    def zfill(r, carry):
        for j in range(D // 16):
            g0[r, pl.ds(j * 16, 16)] = zero16
        return carry

    lax.fori_loop(0, K, zfill, 0)
    for t in range(RPT // K):
        pltpu.sync_copy(g0, acc.at[pl.ds(sid * RPT + t * K, K)])

    @pl.when(sid == NS - 1)
    def _():
        pltpu.sync_copy(g0.at[pl.ds(0, 8)], acc.at[pl.ds(HALF, 8)])

    dummy_row = jnp.full((16,), HALF, jnp.int32)
    zero_i = jnp.zeros((16,), jnp.int32)

    def pfill(p, carry):
        sl = pl.ds(p * 16, 16)
        crow[sl] = dummy_row
        ccol[sl] = zero_i
        cval[sl] = zero16
        return carry

    lax.fori_loop(0, CAP // 16, pfill, 0)

    def issue_gather(c, b):
        pltpu.async_copy(out_hbm.at[ccol.at[pl.ds(c * K, K)]], gbuf[b],
                         semg[b])

    def wait_gather(c, b):
        pltpu.make_async_copy(out_hbm.at[ccol.at[pl.ds(c * K, K)]], gbuf[b],
                              semg[b]).wait()

    def issue_scatter(c, b):
        for g in range(K // 16):
            ridx = crow[pl.ds(c * K + g * 16, 16)]
            pltpu.async_copy(sbuf[b].at[pl.ds(g * 16, 16)], acc.at[ridx],
                             sems[b], add=True)

    def wait_scatter(b):
        for g in range(K // 16):
            pltpu.make_async_copy(sbuf[b].at[pl.ds(g * 16, 16)],
                                  acc.at[dummy_row], sems[b]).wait()

    def drain(cnt):
        nr = (cnt + (RING * K - 1)) // (RING * K)

        @pl.when(nr > 0)
        def _():
            for b in range(RING):
                issue_gather(b, b)

        def round_body(i, carry):
            for b in range(RING):
                c = RING * i + b
                wait_gather(c, b)

                @pl.when(i > 0)
                def _(b=b):
                    wait_scatter(b)

                gb, sb = gbuf[b], sbuf[b]

                def scale_group(g, carry2, c=c, gb=gb, sb=sb):
                    val16 = cval[pl.ds(c * K + g * 16, 16)]
                    for l in range(16):
                        v = jnp.full((16,), val16[l], jnp.float32)
                        e = g * 16 + l
                        for j in range(D // 16):
                            sl = pl.ds(j * 16, 16)
                            sb[e, sl] = gb[e, sl] * v
                    return carry2

                lax.fori_loop(0, K // 16, scale_group, 0)

                @pl.when(i + 1 < nr)
                def _(b=b, c=c):
                    issue_gather(c + RING, b)

                issue_scatter(c, b)
            return carry

        lax.fori_loop(0, nr, round_body, 0)

        @pl.when(nr > 0)
        def _():
            for b in range(RING):
                wait_scatter(b)

        lax.fori_loop(0, CAP // 16, pfill, 0)

    ebase = sid * EPT

    def issue_stage(t, p):
        sl = pl.ds(ebase + t * STAGE, STAGE)
        rr_, rc_, rv_ = rstage[p]
        pltpu.async_copy(row_hbm.at[sl], rr_, semr[p])
        pltpu.async_copy(col_hbm.at[sl], rc_, semr[p])
        pltpu.async_copy(val_hbm.at[sl], rv_, semr[p])

    def wait_stage(t, p):
        sl = pl.ds(ebase + t * STAGE, STAGE)
        rr_, rc_, rv_ = rstage[p]
        pltpu.make_async_copy(row_hbm.at[sl], rr_, semr[p]).wait()
        pltpu.make_async_copy(col_hbm.at[sl], rc_, semr[p]).wait()
        pltpu.make_async_copy(val_hbm.at[sl], rv_, semr[p]).wait()

    def scan_groups(rr_, rc_, rv_, cnt):
        def group_body(g, cnt):
            sl = pl.ds(g * 16, 16)
            rr16 = rr_[sl] - lo
            mask = (rr16 >= 0) & (rr16 < HALF)
            cs = plsc.cumsum(mask.astype(jnp.int32))
            pos = cnt + cs - 1
            plsc.store_scatter(crow, [pos], rr16, mask=mask)
            plsc.store_scatter(ccol, [pos], rc_[sl], mask=mask)
            plsc.store_scatter(cval, [pos], rv_[sl], mask=mask)
            return cnt + cs[15]

        return lax.fori_loop(0, GPS, group_body, cnt)

    for p in range(SRING - 1):
        issue_stage(p, p)

    def stage_round(tr, cnt):
        for p in range(SRING):
            t = SRING * tr + p
            wait_stage(t, p)
            rr_, rc_, rv_ = rstage[p]
            cnt = scan_groups(rr_, rc_, rv_, cnt)

            @pl.when(t + (SRING - 1) < NSTAGE)
            def _(t=t, pn=(p + SRING - 1) % SRING):
                issue_stage(t + (SRING - 1), pn)

            full = cnt > DRAIN_AT

            @pl.when(full)
            def _(cnt=cnt):
                drain(cnt)

            cnt = jnp.where(full, 0, cnt)
        return cnt

    cnt = lax.fori_loop(0, NSTAGE // SRING, stage_round, jnp.int32(0))
    for t in range(NSTAGE - NSTAGE % SRING, NSTAGE):
        p = t % SRING
        wait_stage(t, p)
        rr_, rc_, rv_ = rstage[p]
        cnt = scan_groups(rr_, rc_, rv_, cnt)

    drain(cnt)

    plsc.subcore_barrier()

    pltpu.sync_copy(acc.at[pl.ds(sid * RPT, RPT)],
                    res_hbm.at[pl.ds(lo + sid * RPT, RPT)])


_spmm = functools.partial(
    pl.kernel,
    out_type=jax.ShapeDtypeStruct((NC * HALF, D), jnp.float32),
    mesh=plsc.VectorSubcoreMesh(core_axis_name="c", subcore_axis_name="s"),
    compiler_params=pltpu.CompilerParams(needs_layout_passes=False),
    scratch_types=(
        [
            pltpu.VMEM((CAP,), jnp.int32),
            pltpu.VMEM((CAP,), jnp.int32),
            pltpu.VMEM((CAP,), jnp.float32),
        ]
        + [pltpu.VMEM((STAGE,), dt)
           for _ in range(SRING)
           for dt in (jnp.int32, jnp.int32, jnp.float32)]
        + [pltpu.VMEM((K, D), jnp.float32) for _ in range(2 * RING)]
        + [pltpu.VMEM_SHARED((ACC_H, D), jnp.float32)]
        + [pltpu.SemaphoreType.DMA] * (SRING + 2 * RING)
    ),
)(_spmm_body)


def kernel(x, adj_indices, adj_values, W, b):
    out = _linear(x, W.T, b.reshape(1, D))
    res = _spmm(out, adj_indices[0], adj_indices[1], adj_values)
    return res[:N]
